# baseline (speedup 1.0000x reference)
import functools

import numpy as np
import jax
import jax.numpy as jnp
from jax.experimental import pallas as pl
from jax.experimental.pallas import tpu as pltpu

ALPHA = 0.5
THETA = 0.5


def _pick_bm(n):
    for bm in (1000, 512, 256, 200, 128, 64, 40, 32, 16, 8):
        if n % bm == 0:
            return bm
    return n


def _stats_kernel(x_ref, sum_ref, sq_ref):
    m = pl.program_id(1)
    xb = x_ref[0]
    s = jnp.sum(xb, axis=0, keepdims=True)
    q = jnp.sum(xb * xb, axis=0, keepdims=True)

    @pl.when(m == 0)
    def _():
        sum_ref[0] = s
        sq_ref[0] = q

    @pl.when(m > 0)
    def _():
        sum_ref[0] += s
        sq_ref[0] += q


def _input_kernel(x_ref, sum_ref, sq_ref, gam_ref, bet_ref, w_ref, b_ref,
                  out_ref, *, n_rows):
    xb = x_ref[0]
    mean = sum_ref[0] / n_rows
    var = sq_ref[0] / n_rows - mean * mean
    scale = jax.lax.rsqrt(var + 1e-5) * gam_ref[0]
    shift = bet_ref[0] - mean * scale
    xn = xb * scale + shift
    y = jnp.dot(xn, w_ref[0], preferred_element_type=jnp.float32) + b_ref[0]
    out_ref[0] = jnp.where(y >= 0, y, 0.01 * y)


def _quant_l0_kernel(net_ref, x0_ref, w_ref, q_ref, scl_ref, h1_ref, *,
                     bm, n):
    m = pl.program_id(1)
    blk = net_ref[0]
    rmax = jnp.max(jnp.abs(blk), axis=-1, keepdims=True)
    rmax = jnp.maximum(rmax, 1e-30)
    scl = (rmax * (1.0 / 127.0)).astype(jnp.bfloat16)
    q = jnp.round(blk * (1.0 / scl.astype(jnp.float32)))
    q_ref[0] = jnp.clip(q, -127.0, 127.0).astype(jnp.int8)
    scl_ref[0] = jnp.broadcast_to(scl, (blk.shape[0], scl_ref.shape[-1]))
    x0c = x0_ref[0][:n].astype(jnp.bfloat16)
    x0t = x0_ref[0, pl.ds(m * bm, bm), :]
    s = (1.0 - ALPHA) * jnp.dot(blk.astype(jnp.bfloat16), x0c,
                                preferred_element_type=jnp.float32) + ALPHA * x0t
    hn = jnp.dot(s.astype(jnp.bfloat16), w_ref[0, 0],
                 preferred_element_type=jnp.float32)
    h1_ref[0] = jnp.maximum(hn, 0.0)


def _prop_kernel(q_ref, scl_ref, x0_ref, h1_ref, w_ref, out_ref, scr_ref,
                 hq_ref, cm_ref, *, bm, n):
    l = pl.program_id(1)
    m = pl.program_id(2)

    @pl.when((l == 0) & (m == 0))
    def _():
        scr_ref[0] = h1_ref[0]

    @pl.when(m == 0)
    def _():
        h = scr_ref[l % 2]
        cm = jnp.maximum(jnp.max(jnp.abs(h[:n]), axis=0, keepdims=True), 1e-30)
        cm_ref[...] = cm
        hq_ref[...] = jnp.round(h * (127.0 / cm)).astype(jnp.int8)

    x0t = x0_ref[0, pl.ds(m * bm, bm), :]
    si = jax.lax.dot_general(q_ref[0], hq_ref[:n],
                             (((1,), (0,)), ((), ())),
                             preferred_element_type=jnp.int32)
    s = (si.astype(jnp.float32) * scl_ref[0].astype(jnp.float32)
         * (cm_ref[...] * (1.0 / 127.0)))
    s = (1.0 - ALPHA) * s + ALPHA * x0t
    hn = jnp.dot(s.astype(jnp.bfloat16), w_ref[0, 0],
                 preferred_element_type=jnp.float32)
    hn = jnp.maximum(hn, 0.0)
    scr_ref[(l + 1) % 2, pl.ds(m * bm, bm), :] = hn
    out_ref[0] = hn


def _cls_kernel(hid_ref, wcat_ref, bcat_ref, wcls_ref, bcls_ref, out_ref, *,
                dim):
    e = (jnp.dot(hid_ref[0], wcat_ref[:dim], preferred_element_type=jnp.float32)
         + jnp.dot(hid_ref[1], wcat_ref[dim:], preferred_element_type=jnp.float32)
         + bcat_ref[0])
    e = jnp.where(e >= 0, e, 0.01 * e)
    out_ref[...] = jnp.dot(e, wcls_ref[...],
                           preferred_element_type=jnp.float32) + bcls_ref[0]


def kernel(x, net, bn_gamma, bn_beta, W_in, b_in, W_gcn, W_cat, b_cat, W_cls,
           b_cls):
    nv, n, _ = x.shape
    dim = W_in.shape[-1]
    nl = W_gcn.shape[1]
    nt = W_cls.shape[-1]
    bm = _pick_bm(n)
    nm = n // bm
    rup = lambda a, b: -(-a // b) * b
    bmb = bm
    nmb = nm
    npadb = n
    nmp = max(1, -(-n // 2560))
    bmp = rup(-(-n // nmp), 32)
    npad = nmp * bmp
    f32 = jnp.float32

    colsum, colsq = pl.pallas_call(
        _stats_kernel,
        grid=(nv, nm),
        in_specs=[pl.BlockSpec((1, bm, n), lambda v, m: (v, m, 0))],
        out_specs=[pl.BlockSpec((1, 1, n), lambda v, m: (v, 0, 0)),
                   pl.BlockSpec((1, 1, n), lambda v, m: (v, 0, 0))],
        out_shape=[jax.ShapeDtypeStruct((nv, 1, n), f32),
                   jax.ShapeDtypeStruct((nv, 1, n), f32)],
    )(x)

    x0 = pl.pallas_call(
        functools.partial(_input_kernel, n_rows=n),
        grid=(nv, nmb),
        in_specs=[
            pl.BlockSpec((1, bmb, n), lambda v, m: (v, m, 0)),
            pl.BlockSpec((1, 1, n), lambda v, m: (v, 0, 0)),
            pl.BlockSpec((1, 1, n), lambda v, m: (v, 0, 0)),
            pl.BlockSpec((1, 1, n), lambda v, m: (v, 0, 0)),
            pl.BlockSpec((1, 1, n), lambda v, m: (v, 0, 0)),
            pl.BlockSpec((1, n, dim), lambda v, m: (v, 0, 0)),
            pl.BlockSpec((1, 1, dim), lambda v, m: (v, 0, 0)),
        ],
        out_specs=pl.BlockSpec((1, bmb, dim), lambda v, m: (v, m, 0)),
        out_shape=jax.ShapeDtypeStruct((nv, npadb, dim), f32),
    )(x, colsum, colsq, bn_gamma.reshape(nv, 1, n), bn_beta.reshape(nv, 1, n),
      W_in, b_in.reshape(nv, 1, dim))

    betas = np.log(THETA / np.arange(1, nl + 1) + 1.0).astype(np.float32)
    eye = jnp.eye(dim, dtype=f32)
    W_eff = ((1.0 - betas)[None, :, None, None] * eye[None, None]
             + betas[None, :, None, None] * W_gcn).astype(jnp.bfloat16)

    bmc = min(512, bmp)
    nmc = -(-npad // bmc)
    npadc = nmc * bmc
    net_q, net_scl, h1 = pl.pallas_call(
        functools.partial(_quant_l0_kernel, bm=bmc, n=n),
        grid=(nv, nmc),
        in_specs=[
            pl.BlockSpec((1, bmc, n), lambda v, m: (v, m, 0)),
            pl.BlockSpec((1, npadb, dim), lambda v, m: (v, 0, 0)),
            pl.BlockSpec((1, 1, dim, dim), lambda v, m: (v, 0, 0, 0)),
        ],
        out_specs=[pl.BlockSpec((1, bmc, n), lambda v, m: (v, m, 0)),
                   pl.BlockSpec((1, bmc, 128), lambda v, m: (v, m, 0)),
                   pl.BlockSpec((1, bmc, dim), lambda v, m: (v, m, 0))],
        out_shape=[jax.ShapeDtypeStruct((nv, npad, n), jnp.int8),
                   jax.ShapeDtypeStruct((nv, npad, 128), jnp.bfloat16),
                   jax.ShapeDtypeStruct((nv, npadc, dim), f32)],
    )(net, x0, W_eff)

    hidden = pl.pallas_call(
        functools.partial(_prop_kernel, bm=bmp, n=n),
        grid=(nv, nl - 1, nmp),
        in_specs=[
            pl.BlockSpec((1, bmp, n), lambda v, l, m: (v, m, 0)),
            pl.BlockSpec((1, bmp, 128), lambda v, l, m: (v, m, 0)),
            pl.BlockSpec((1, npad, dim), lambda v, l, m: (v, 0, 0)),
            pl.BlockSpec((1, npad, dim), lambda v, l, m: (v, 0, 0)),
            pl.BlockSpec((1, 1, dim, dim), lambda v, l, m: (v, l + 1, 0, 0)),
        ],
        out_specs=pl.BlockSpec((1, bmp, dim), lambda v, l, m: (v, m, 0)),
        out_shape=jax.ShapeDtypeStruct((nv, npad, dim), f32),
        scratch_shapes=[pltpu.VMEM((2, npad, dim), f32),
                        pltpu.VMEM((npad, dim), jnp.int8),
                        pltpu.VMEM((1, 128), f32)],
    )(net_q, net_scl, x0, h1, W_eff)

    pred = pl.pallas_call(
        functools.partial(_cls_kernel, dim=dim),
        grid=(nm,),
        in_specs=[
            pl.BlockSpec((nv, bm, dim), lambda m: (0, m, 0)),
            pl.BlockSpec((nv * dim, dim), lambda m: (0, 0)),
            pl.BlockSpec((1, dim), lambda m: (0, 0)),
            pl.BlockSpec((dim, nt), lambda m: (0, 0)),
            pl.BlockSpec((1, nt), lambda m: (0, 0)),
        ],
        out_specs=pl.BlockSpec((bm, nt), lambda m: (m, 0)),
        out_shape=jax.ShapeDtypeStruct((n, nt), f32),
    )(hidden, W_cat, b_cat.reshape(1, dim), W_cls, b_cls.reshape(1, nt))
    return pred


# speedup vs baseline: 1.0191x; 1.0191x over previous
import functools

import numpy as np
import jax
import jax.numpy as jnp
from jax.experimental import pallas as pl
from jax.experimental.pallas import tpu as pltpu

ALPHA = 0.5
THETA = 0.5


def _pick_bm(n):
    for bm in (1000, 512, 256, 200, 128, 64, 40, 32, 16, 8):
        if n % bm == 0:
            return bm
    return n


def _stats_kernel(x_ref, sum_ref, sq_ref):
    m = pl.program_id(1)
    xb = x_ref[0]
    s = jnp.sum(xb, axis=0, keepdims=True)
    q = jnp.sum(xb * xb, axis=0, keepdims=True)

    @pl.when(m == 0)
    def _():
        sum_ref[0] = s
        sq_ref[0] = q

    @pl.when(m > 0)
    def _():
        sum_ref[0] += s
        sq_ref[0] += q


def _input_kernel(x_ref, sum_ref, sq_ref, gam_ref, bet_ref, w_ref, b_ref,
                  out_ref, *, n_rows):
    xb = x_ref[0]
    mean = sum_ref[0] / n_rows
    var = sq_ref[0] / n_rows - mean * mean
    scale = jax.lax.rsqrt(var + 1e-5) * gam_ref[0]
    shift = bet_ref[0] - mean * scale
    xn = xb * scale + shift
    y = jnp.dot(xn, w_ref[0], preferred_element_type=jnp.float32) + b_ref[0]
    out_ref[0] = jnp.where(y >= 0, y, 0.01 * y)


def _quant_l0_kernel(net_ref, x0_ref, w_ref, q_ref, scl_ref, h1_ref, *,
                     bm, n):
    m = pl.program_id(1)
    blk = net_ref[0]
    rmax = jnp.max(jnp.abs(blk), axis=-1, keepdims=True)
    rmax = jnp.maximum(rmax, 1e-30)
    q_ref[0] = jnp.round(blk * (127.0 / rmax)).astype(jnp.int8)
    scl_ref[0] = jnp.broadcast_to(rmax * (1.0 / 127.0),
                                  (blk.shape[0], scl_ref.shape[-1]))
    x0c = x0_ref[0][:n].astype(jnp.bfloat16)
    x0t = x0_ref[0, pl.ds(m * bm, bm), :]
    s = (1.0 - ALPHA) * jnp.dot(blk.astype(jnp.bfloat16), x0c,
                                preferred_element_type=jnp.float32) + ALPHA * x0t
    hn = jnp.dot(s.astype(jnp.bfloat16), w_ref[0, 0],
                 preferred_element_type=jnp.float32)
    h1_ref[0] = jnp.maximum(hn, 0.0)


def _prop_kernel(q_ref, scl_ref, x0_ref, h1_ref, w_ref, out_ref, scr_ref,
                 hq_ref, cm_ref, *, bm, n):
    l = pl.program_id(1)
    m = pl.program_id(2)

    @pl.when((l == 0) & (m == 0))
    def _():
        scr_ref[0] = h1_ref[0]

    @pl.when(m == 0)
    def _():
        h = scr_ref[l % 2]
        cm = jnp.maximum(jnp.max(jnp.abs(h[:n]), axis=0, keepdims=True), 1e-30)
        cm_ref[...] = cm
        hq_ref[...] = jnp.round(h * (127.0 / cm)).astype(jnp.int8)

    x0t = x0_ref[0, pl.ds(m * bm, bm), :]
    si = jax.lax.dot_general(q_ref[0], hq_ref[:n],
                             (((1,), (0,)), ((), ())),
                             preferred_element_type=jnp.int32)
    s = si.astype(jnp.float32) * scl_ref[0] * (cm_ref[...] * (1.0 / 127.0))
    s = (1.0 - ALPHA) * s + ALPHA * x0t
    hn = jnp.dot(s.astype(jnp.bfloat16), w_ref[0, 0],
                 preferred_element_type=jnp.float32)
    hn = jnp.maximum(hn, 0.0)
    scr_ref[(l + 1) % 2, pl.ds(m * bm, bm), :] = hn
    out_ref[0] = hn


def _cls_kernel(hid_ref, wcat_ref, bcat_ref, wcls_ref, bcls_ref, out_ref, *,
                dim):
    e = (jnp.dot(hid_ref[0], wcat_ref[:dim], preferred_element_type=jnp.float32)
         + jnp.dot(hid_ref[1], wcat_ref[dim:], preferred_element_type=jnp.float32)
         + bcat_ref[0])
    e = jnp.where(e >= 0, e, 0.01 * e)
    out_ref[...] = jnp.dot(e, wcls_ref[...],
                           preferred_element_type=jnp.float32) + bcls_ref[0]


def kernel(x, net, bn_gamma, bn_beta, W_in, b_in, W_gcn, W_cat, b_cat, W_cls,
           b_cls):
    nv, n, _ = x.shape
    dim = W_in.shape[-1]
    nl = W_gcn.shape[1]
    nt = W_cls.shape[-1]
    bm = _pick_bm(n)
    nm = n // bm
    rup = lambda a, b: -(-a // b) * b
    bmb = bm
    nmb = nm
    npadb = n
    nmp = max(1, -(-n // 2560))
    bmp = rup(-(-n // nmp), 32)
    npad = nmp * bmp
    f32 = jnp.float32

    colsum, colsq = pl.pallas_call(
        _stats_kernel,
        grid=(nv, nm),
        in_specs=[pl.BlockSpec((1, bm, n), lambda v, m: (v, m, 0))],
        out_specs=[pl.BlockSpec((1, 1, n), lambda v, m: (v, 0, 0)),
                   pl.BlockSpec((1, 1, n), lambda v, m: (v, 0, 0))],
        out_shape=[jax.ShapeDtypeStruct((nv, 1, n), f32),
                   jax.ShapeDtypeStruct((nv, 1, n), f32)],
    )(x)

    x0 = pl.pallas_call(
        functools.partial(_input_kernel, n_rows=n),
        grid=(nv, nmb),
        in_specs=[
            pl.BlockSpec((1, bmb, n), lambda v, m: (v, m, 0)),
            pl.BlockSpec((1, 1, n), lambda v, m: (v, 0, 0)),
            pl.BlockSpec((1, 1, n), lambda v, m: (v, 0, 0)),
            pl.BlockSpec((1, 1, n), lambda v, m: (v, 0, 0)),
            pl.BlockSpec((1, 1, n), lambda v, m: (v, 0, 0)),
            pl.BlockSpec((1, n, dim), lambda v, m: (v, 0, 0)),
            pl.BlockSpec((1, 1, dim), lambda v, m: (v, 0, 0)),
        ],
        out_specs=pl.BlockSpec((1, bmb, dim), lambda v, m: (v, m, 0)),
        out_shape=jax.ShapeDtypeStruct((nv, npadb, dim), f32),
    )(x, colsum, colsq, bn_gamma.reshape(nv, 1, n), bn_beta.reshape(nv, 1, n),
      W_in, b_in.reshape(nv, 1, dim))

    betas = np.log(THETA / np.arange(1, nl + 1) + 1.0).astype(np.float32)
    eye = jnp.eye(dim, dtype=f32)
    W_eff = ((1.0 - betas)[None, :, None, None] * eye[None, None]
             + betas[None, :, None, None] * W_gcn).astype(jnp.bfloat16)

    bmc = min(512, bmp)
    nmc = -(-npad // bmc)
    npadc = nmc * bmc
    net_q, net_scl, h1 = pl.pallas_call(
        functools.partial(_quant_l0_kernel, bm=bmc, n=n),
        grid=(nv, nmc),
        in_specs=[
            pl.BlockSpec((1, bmc, n), lambda v, m: (v, m, 0)),
            pl.BlockSpec((1, npadb, dim), lambda v, m: (v, 0, 0)),
            pl.BlockSpec((1, 1, dim, dim), lambda v, m: (v, 0, 0, 0)),
        ],
        out_specs=[pl.BlockSpec((1, bmc, n), lambda v, m: (v, m, 0)),
                   pl.BlockSpec((1, bmc, 128), lambda v, m: (v, m, 0)),
                   pl.BlockSpec((1, bmc, dim), lambda v, m: (v, m, 0))],
        out_shape=[jax.ShapeDtypeStruct((nv, npad, n), jnp.int8),
                   jax.ShapeDtypeStruct((nv, npad, 128), f32),
                   jax.ShapeDtypeStruct((nv, npadc, dim), f32)],
    )(net, x0, W_eff)

    hidden = pl.pallas_call(
        functools.partial(_prop_kernel, bm=bmp, n=n),
        grid=(nv, nl - 1, nmp),
        in_specs=[
            pl.BlockSpec((1, bmp, n), lambda v, l, m: (v, m, 0)),
            pl.BlockSpec((1, bmp, 128), lambda v, l, m: (v, m, 0)),
            pl.BlockSpec((1, npad, dim), lambda v, l, m: (v, 0, 0)),
            pl.BlockSpec((1, npad, dim), lambda v, l, m: (v, 0, 0)),
            pl.BlockSpec((1, 1, dim, dim), lambda v, l, m: (v, l + 1, 0, 0)),
        ],
        out_specs=pl.BlockSpec((1, bmp, dim), lambda v, l, m: (v, m, 0)),
        out_shape=jax.ShapeDtypeStruct((nv, npad, dim), f32),
        scratch_shapes=[pltpu.VMEM((2, npad, dim), f32),
                        pltpu.VMEM((npad, dim), jnp.int8),
                        pltpu.VMEM((1, 128), f32)],
    )(net_q, net_scl, x0, h1, W_eff)

    pred = pl.pallas_call(
        functools.partial(_cls_kernel, dim=dim),
        grid=(nm,),
        in_specs=[
            pl.BlockSpec((nv, bm, dim), lambda m: (0, m, 0)),
            pl.BlockSpec((nv * dim, dim), lambda m: (0, 0)),
            pl.BlockSpec((1, dim), lambda m: (0, 0)),
            pl.BlockSpec((dim, nt), lambda m: (0, 0)),
            pl.BlockSpec((1, nt), lambda m: (0, 0)),
        ],
        out_specs=pl.BlockSpec((bm, nt), lambda m: (m, 0)),
        out_shape=jax.ShapeDtypeStruct((n, nt), f32),
    )(hidden, W_cat, b_cat.reshape(1, dim), W_cls, b_cls.reshape(1, nt))
    return pred
